# baseline (speedup 1.0000x reference)
import functools

import jax
import jax.numpy as jnp
import numpy as np
from jax import lax
from jax.experimental import pallas as pl
from jax.experimental.pallas import tpu as pltpu
from jax.experimental.pallas import tpu_sc as plsc

_B, _D, _L, _E = 64, 64, 1024, 1024
_N = _B * _D * _L
_NV = _B * _L

_SRANK = np.array([0, 7, 3, 5, 1, 6, 2, 4], dtype=np.int64)
_EIDX = np.arange(_E, dtype=np.int64)
_PERM = _SRANK[_EIDX & 7] * (_E // 8) + (_EIDX >> 3)

_BB = 16


def _argmin_body(z_ref, w_ref, idx_ref, loss_ref):
    w = w_ref[...]
    w2 = -2.0 * w
    wsq = jnp.sum(w * w, axis=1, keepdims=True)
    for i in range(_BB):
        z = z_ref[i]
        s2 = jax.lax.dot_general(
            w2, z, (((1,), (0,)), ((), ())), preferred_element_type=jnp.float32
        )
        zsq = jnp.sum(z * z, axis=0, keepdims=True)
        d = (zsq + wsq) + s2
        idx_ref[i] = jnp.argmin(d, axis=0).reshape(1, _L)
        loss_ref[i] = jnp.sum(jnp.min(d, axis=0)).reshape(1, 1)


_SC_CHUNK = 512
_DP = 128


def _sc_gather(table, idx_flat):
    nv = idx_flat.shape[0]
    info = plsc.get_sparse_core_info()
    nw = info.num_cores * info.num_subcores
    b_per_w = nv // nw
    n_chunks = b_per_w // _SC_CHUNK
    mesh = plsc.VectorSubcoreMesh(core_axis_name="c", subcore_axis_name="s")

    scratch = []
    for _ in range(n_chunks):
        scratch += [
            pltpu.VMEM((_SC_CHUNK,), jnp.int32),
            pltpu.VMEM((_SC_CHUNK, _DP), jnp.float32),
            pltpu.SemaphoreType.DMA,
            pltpu.SemaphoreType.DMA,
        ]

    @functools.partial(
        pl.kernel,
        mesh=mesh,
        out_type=jax.ShapeDtypeStruct((nv, _DP), jnp.float32),
        scratch_types=scratch,
    )
    def k(table_hbm, idx_hbm, out_hbm, *bufs):
        wid = lax.axis_index("s") * info.num_cores + lax.axis_index("c")
        base = wid * b_per_w
        gathers = []
        for i in range(n_chunks):
            idx_v, rows_v, gsem, _ = bufs[4 * i:4 * i + 4]
            off = base + i * _SC_CHUNK
            pltpu.sync_copy(idx_hbm.at[pl.ds(off, _SC_CHUNK)], idx_v)
            gathers.append(pltpu.async_copy(table_hbm.at[idx_v], rows_v, gsem))
        writes = []
        for i in range(n_chunks):
            _, rows_v, _, wsem = bufs[4 * i:4 * i + 4]
            off = base + i * _SC_CHUNK
            gathers[i].wait()
            writes.append(
                pltpu.async_copy(rows_v, out_hbm.at[pl.ds(off, _SC_CHUNK)], wsem)
            )
        for wcopy in writes:
            wcopy.wait()

    return k(table, idx_flat)


_TB = 8


def _transpose_body(qf_ref, out_ref):
    for i in range(_TB):
        out_ref[i] = qf_ref[i][:, :_D].T


def _argmin_tr_body(z_ref, w_ref, qf_ref, idx_ref, loss_ref, q_ref):
    w = w_ref[...]
    w2 = -2.0 * w
    wsq = jnp.sum(w * w, axis=1, keepdims=True)
    for i in range(_BB):
        z = z_ref[i]
        s2 = jax.lax.dot_general(
            w2, z, (((1,), (0,)), ((), ())), preferred_element_type=jnp.float32
        )
        zsq = jnp.sum(z * z, axis=0, keepdims=True)
        d = (zsq + wsq) + s2
        idx_ref[i] = jnp.argmin(d, axis=0).reshape(1, _L)
        loss_ref[i] = jnp.sum(jnp.min(d, axis=0)).reshape(1, 1)
        q_ref[i] = qf_ref[i][:, :_D].T


def _argmin_tr_call(zc, w_perm, qfc):
    return pl.pallas_call(
        _argmin_tr_body,
        grid=(_CB // _BB,),
        in_specs=[
            pl.BlockSpec((_BB, _D, _L), lambda b: (b, 0, 0)),
            pl.BlockSpec((_E, _D), lambda b: (0, 0)),
            pl.BlockSpec((_BB, _L, _DP), lambda b: (b, 0, 0)),
        ],
        out_specs=[
            pl.BlockSpec((_BB, 1, _L), lambda b: (b, 0, 0)),
            pl.BlockSpec((_BB, 1, 1), lambda b: (b, 0, 0)),
            pl.BlockSpec((_BB, _D, _L), lambda b: (b, 0, 0)),
        ],
        out_shape=[
            jax.ShapeDtypeStruct((_CB, 1, _L), jnp.int32),
            jax.ShapeDtypeStruct((_CB, 1, 1), jnp.float32),
            jax.ShapeDtypeStruct((_CB, _D, _L), jnp.float32),
        ],
        compiler_params=pltpu.CompilerParams(
            dimension_semantics=("parallel",),
        ),
    )(zc, w_perm, qfc)

_CHUNKS = 4
_CB = _B // _CHUNKS


def _argmin_call(zc, w_perm):
    return pl.pallas_call(
        _argmin_body,
        grid=(_CB // _BB,),
        in_specs=[
            pl.BlockSpec((_BB, _D, _L), lambda b: (b, 0, 0)),
            pl.BlockSpec((_E, _D), lambda b: (0, 0)),
        ],
        out_specs=[
            pl.BlockSpec((_BB, 1, _L), lambda b: (b, 0, 0)),
            pl.BlockSpec((_BB, 1, 1), lambda b: (b, 0, 0)),
        ],
        out_shape=[
            jax.ShapeDtypeStruct((_CB, 1, _L), jnp.int32),
            jax.ShapeDtypeStruct((_CB, 1, 1), jnp.float32),
        ],
        compiler_params=pltpu.CompilerParams(
            dimension_semantics=("parallel",),
        ),
    )(zc, w_perm)


def _transpose_call(qfc):
    return pl.pallas_call(
        _transpose_body,
        grid=(_CB // _TB,),
        in_specs=[pl.BlockSpec((_TB, _L, _DP), lambda b: (b, 0, 0))],
        out_specs=pl.BlockSpec((_TB, _D, _L), lambda b: (b, 0, 0)),
        out_shape=jax.ShapeDtypeStruct((_CB, _D, _L), jnp.float32),
        compiler_params=pltpu.CompilerParams(
            dimension_semantics=("parallel",),
        ),
    )(qfc)


@jax.jit
def kernel(z, W):
    w_perm = W[_PERM]
    w_pad = jnp.zeros((_E, _DP), jnp.float32).at[:, :_D].set(w_perm)
    qs, losses = [], []
    idx_c, loss_c = _argmin_call(
        lax.slice_in_dim(z, 0, _CB, axis=0), w_perm)
    losses.append(loss_c)
    qf_prev = _sc_gather(w_pad, idx_c.reshape(_CB * _L))
    for c in range(1, _CHUNKS):
        zc = lax.slice_in_dim(z, c * _CB, (c + 1) * _CB, axis=0)
        idx_c, loss_c, q_prev = _argmin_tr_call(
            zc, w_perm, qf_prev.reshape(_CB, _L, _DP))
        qs.append(q_prev)
        losses.append(loss_c)
        qf_prev = _sc_gather(w_pad, idx_c.reshape(_CB * _L))
    qs.append(_transpose_call(qf_prev.reshape(_CB, _L, _DP)))
    q = jnp.concatenate(qs, axis=0)
    vq_loss = jnp.sum(jnp.stack(losses)) / _N
    return q, vq_loss, 0.25 * vq_loss


# speedup vs baseline: 1.3092x; 1.3092x over previous
import functools

import jax
import jax.numpy as jnp
import numpy as np
from jax import lax
from jax.experimental import pallas as pl
from jax.experimental.pallas import tpu as pltpu
from jax.experimental.pallas import tpu_sc as plsc

_B, _D, _L, _E = 64, 64, 1024, 1024
_N = _B * _D * _L
_NV = _B * _L

_SRANK = np.array([0, 7, 3, 5, 1, 6, 2, 4], dtype=np.int64)
_EIDX = np.arange(_E, dtype=np.int64)
_PERM = _SRANK[_EIDX & 7] * (_E // 8) + (_EIDX >> 3)

_BB = 16


def _argmin_body(z_ref, w_ref, idx_ref, loss_ref):
    w = w_ref[...]
    w2 = -2.0 * w
    wsq = jnp.sum(w * w, axis=1, keepdims=True)
    for i in range(_BB):
        z = z_ref[i]
        s2 = jax.lax.dot_general(
            w2, z, (((1,), (0,)), ((), ())), preferred_element_type=jnp.float32
        )
        zsq = jnp.sum(z * z, axis=0, keepdims=True)
        d = (zsq + wsq) + s2
        idx_ref[i] = jnp.argmin(d, axis=0).reshape(1, _L)
        loss_ref[i] = jnp.sum(jnp.min(d, axis=0)).reshape(1, 1)


_SC_CHUNK = 512
_DP = 128


def _sc_gather(table, idx_flat):
    nv = idx_flat.shape[0]
    info = plsc.get_sparse_core_info()
    nw = info.num_cores * info.num_subcores
    b_per_w = nv // nw
    n_chunks = b_per_w // _SC_CHUNK
    mesh = plsc.VectorSubcoreMesh(core_axis_name="c", subcore_axis_name="s")

    scratch = []
    for _ in range(n_chunks):
        scratch += [
            pltpu.VMEM((_SC_CHUNK,), jnp.int32),
            pltpu.VMEM((_SC_CHUNK, _DP), jnp.float32),
            pltpu.SemaphoreType.DMA,
            pltpu.SemaphoreType.DMA,
        ]

    @functools.partial(
        pl.kernel,
        mesh=mesh,
        out_type=jax.ShapeDtypeStruct((nv, _DP), jnp.float32),
        scratch_types=scratch,
    )
    def k(table_hbm, idx_hbm, out_hbm, *bufs):
        wid = lax.axis_index("s") * info.num_cores + lax.axis_index("c")
        base = wid * b_per_w
        gathers = []
        for i in range(n_chunks):
            idx_v, rows_v, gsem, _ = bufs[4 * i:4 * i + 4]
            off = base + i * _SC_CHUNK
            pltpu.sync_copy(idx_hbm.at[pl.ds(off, _SC_CHUNK)], idx_v)
            gathers.append(pltpu.async_copy(table_hbm.at[idx_v], rows_v, gsem))
        writes = []
        for i in range(n_chunks):
            _, rows_v, _, wsem = bufs[4 * i:4 * i + 4]
            off = base + i * _SC_CHUNK
            gathers[i].wait()
            writes.append(
                pltpu.async_copy(rows_v, out_hbm.at[pl.ds(off, _SC_CHUNK)], wsem)
            )
        for wcopy in writes:
            wcopy.wait()

    return k(table, idx_flat)


_TB = 8


def _transpose_body(qf_ref, out_ref):
    for i in range(_TB):
        out_ref[i] = qf_ref[i][:, :_D].T


_CHUNKS = 4
_CB = _B // _CHUNKS


def _argmin_call(zc, w_perm):
    return pl.pallas_call(
        _argmin_body,
        grid=(_CB // _BB,),
        in_specs=[
            pl.BlockSpec((_BB, _D, _L), lambda b: (b, 0, 0)),
            pl.BlockSpec((_E, _D), lambda b: (0, 0)),
        ],
        out_specs=[
            pl.BlockSpec((_BB, 1, _L), lambda b: (b, 0, 0)),
            pl.BlockSpec((_BB, 1, 1), lambda b: (b, 0, 0)),
        ],
        out_shape=[
            jax.ShapeDtypeStruct((_CB, 1, _L), jnp.int32),
            jax.ShapeDtypeStruct((_CB, 1, 1), jnp.float32),
        ],
        compiler_params=pltpu.CompilerParams(
            dimension_semantics=("parallel",),
        ),
    )(zc, w_perm)


def _transpose_call(qfc):
    return pl.pallas_call(
        _transpose_body,
        grid=(_CB // _TB,),
        in_specs=[pl.BlockSpec((_TB, _L, _DP), lambda b: (b, 0, 0))],
        out_specs=pl.BlockSpec((_TB, _D, _L), lambda b: (b, 0, 0)),
        out_shape=jax.ShapeDtypeStruct((_CB, _D, _L), jnp.float32),
        compiler_params=pltpu.CompilerParams(
            dimension_semantics=("parallel",),
        ),
    )(qfc)


@jax.jit
def kernel(z, W):
    w_perm = W[_PERM]
    w_pad = jnp.zeros((_E, _DP), jnp.float32).at[:, :_D].set(w_perm)
    qs, losses = [], []
    for c in range(_CHUNKS):
        zc = lax.slice_in_dim(z, c * _CB, (c + 1) * _CB, axis=0)
        idx_c, loss_c = _argmin_call(zc, w_perm)
        qf_c = _sc_gather(w_pad, idx_c.reshape(_CB * _L))
        qs.append(_transpose_call(qf_c.reshape(_CB, _L, _DP)))
        losses.append(loss_c)
    q = jnp.concatenate(qs, axis=0)
    vq_loss = jnp.sum(jnp.stack(losses)) / _N
    return q, vq_loss, 0.25 * vq_loss
